# baseline (speedup 1.0000x reference)
import jax
import jax.numpy as jnp
from jax import lax
from jax.experimental import pallas as pl
from jax.experimental.pallas import tpu as pltpu
from jax.experimental.pallas import tpu_sc as plsc

B, T_HALF, D = 4, 1024, 1024
T = 2 * T_HALF
NW = 32
ROWS_PER_W = T // NW
SUB = 8
NG = ROWS_PER_W // SUB
NP = 3
LANES = 16
HALF_VREGS = 32


def _sc_body(seq_a, seq_b, emb, out, *scratch):
    tabs = scratch[0:NP]
    bufs = tuple(scratch[NP + p * B:NP + (p + 1) * B] for p in range(NP))
    base = NP + NP * B
    sem_t = scratch[base:base + NP]
    sem_l = tuple(scratch[base + NP + p * B:base + NP + (p + 1) * B]
                  for p in range(NP))
    base2 = base + NP + NP * B
    sem_o = tuple(scratch[base2 + p * B:base2 + (p + 1) * B]
                  for p in range(NP))

    cid = lax.axis_index("c")
    sid = lax.axis_index("s")
    wid = sid * 2 + cid
    half = wid // 16
    r0 = (wid % 16) * ROWS_PER_W
    g0 = wid * ROWS_PER_W

    def load_group(g):
        p = g % NP
        pltpu.async_copy(emb.at[pl.ds(g0 + SUB * g, SUB), :],
                         tabs[p], sem_t[p])
        for b in range(B):
            @pl.when(half == 0)
            def _():
                pltpu.async_copy(seq_a.at[b, pl.ds(r0 + SUB * g, SUB), :],
                                 bufs[p][b], sem_l[p][b])

            @pl.when(half == 1)
            def _():
                pltpu.async_copy(seq_b.at[b, pl.ds(r0 + SUB * g, SUB), :],
                                 bufs[p][b], sem_l[p][b])

    def wait_group_loads(g):
        p = g % NP
        pltpu.make_async_copy(emb.at[pl.ds(g0 + SUB * g, SUB), :],
                              tabs[p], sem_t[p]).wait()
        for b in range(B):
            pltpu.make_async_copy(seq_a.at[b, pl.ds(r0 + SUB * g, SUB), :],
                                  bufs[p][b], sem_l[p][b]).wait()

    def store_group(g):
        p = g % NP
        for b in range(B):
            pltpu.async_copy(bufs[p][b],
                             out.at[b, pl.ds(g0 + SUB * g, SUB), :],
                             sem_o[p][b])

    def wait_group_stores(g):
        p = g % NP
        for b in range(B):
            pltpu.make_async_copy(bufs[p][b],
                                  out.at[b, pl.ds(g0 + SUB * g, SUB), :],
                                  sem_o[p][b]).wait()

    def compute_group(g):
        p = g % NP
        tab = tabs[p]
        gb = bufs[p]

        def row(r, carry):
            for h in range(D // (HALF_VREGS * LANES)):
                hbase = h * HALF_VREGS * LANES
                tvs = [tab[r, pl.ds(hbase + j * LANES, LANES)]
                       for j in range(HALF_VREGS)]
                for b in range(B):
                    buf = gb[b]
                    for j in range(HALF_VREGS):
                        sl = pl.ds(hbase + j * LANES, LANES)
                        buf[r, sl] = buf[r, sl] + tvs[j]
            return carry

        lax.fori_loop(0, SUB, row, 0)

    load_group(0)
    for g in range(NG):
        if g + 1 < NG:
            if g >= 2:
                wait_group_stores(g - 2)
            load_group(g + 1)
        wait_group_loads(g)
        compute_group(g)
        store_group(g)

    for g in range(max(0, NG - NP), NG):
        wait_group_stores(g)


def kernel(seq_a, seq_b, emb_table):
    mesh = plsc.VectorSubcoreMesh(core_axis_name="c", subcore_axis_name="s")
    f = pl.kernel(
        _sc_body,
        out_type=jax.ShapeDtypeStruct((B, T, D), jnp.float32),
        mesh=mesh,
        scratch_types=(
            [pltpu.VMEM((SUB, D), jnp.float32) for _ in range(NP)]
            + [pltpu.VMEM((SUB, D), jnp.float32) for _ in range(NP * B)]
            + [pltpu.SemaphoreType.DMA for _ in range(NP + 2 * NP * B)]
        ),
    )
    return f(seq_a, seq_b, emb_table)


# speedup vs baseline: 1.0021x; 1.0021x over previous
import jax
import jax.numpy as jnp
from jax import lax
from jax.experimental import pallas as pl
from jax.experimental.pallas import tpu as pltpu
from jax.experimental.pallas import tpu_sc as plsc

B, T_HALF, D = 4, 1024, 1024
T = 2 * T_HALF
NW = 32
ROWS_PER_W = T // NW
SUB = 8
NG = ROWS_PER_W // SUB
NP = 3
LANES = 16
HALF_VREGS = 32


def _sc_body(seq_a, seq_b, emb, out, *scratch):
    tabs = scratch[0:NP]
    bufs = tuple(scratch[NP + p * B:NP + (p + 1) * B] for p in range(NP))
    base = NP + NP * B
    sem_t = scratch[base:base + NP]
    sem_l = tuple(scratch[base + NP + p * B:base + NP + (p + 1) * B]
                  for p in range(NP))
    base2 = base + NP + NP * B
    sem_o = tuple(scratch[base2 + p * B:base2 + (p + 1) * B]
                  for p in range(NP))

    cid = lax.axis_index("c")
    sid = lax.axis_index("s")
    wid = sid * 2 + cid
    half = wid // 16
    r0 = (wid % 16) * ROWS_PER_W
    g0 = wid * ROWS_PER_W

    def load_group(g):
        p = g % NP
        pltpu.async_copy(emb.at[pl.ds(g0 + SUB * g, SUB), :],
                         tabs[p], sem_t[p])
        for b in range(B):
            @pl.when(half == 0)
            def _():
                pltpu.async_copy(seq_a.at[b, pl.ds(r0 + SUB * g, SUB), :],
                                 bufs[p][b], sem_l[p][b])

            @pl.when(half == 1)
            def _():
                pltpu.async_copy(seq_b.at[b, pl.ds(r0 + SUB * g, SUB), :],
                                 bufs[p][b], sem_l[p][b])

    def wait_group_loads(g):
        p = g % NP
        pltpu.make_async_copy(emb.at[pl.ds(g0 + SUB * g, SUB), :],
                              tabs[p], sem_t[p]).wait()
        for b in range(B):
            pltpu.make_async_copy(seq_a.at[b, pl.ds(r0 + SUB * g, SUB), :],
                                  bufs[p][b], sem_l[p][b]).wait()

    def store_group(g):
        p = g % NP
        for b in range(B):
            pltpu.async_copy(bufs[p][b],
                             out.at[b, pl.ds(g0 + SUB * g, SUB), :],
                             sem_o[p][b])

    def wait_group_stores(g):
        p = g % NP
        for b in range(B):
            pltpu.make_async_copy(bufs[p][b],
                                  out.at[b, pl.ds(g0 + SUB * g, SUB), :],
                                  sem_o[p][b]).wait()

    def compute_group(g):
        p = g % NP
        tab = tabs[p]
        gb = bufs[p]

        def row(r, carry):
            for h in range(D // (HALF_VREGS * LANES)):
                hbase = h * HALF_VREGS * LANES
                tvs = [tab[r, pl.ds(hbase + j * LANES, LANES)]
                       for j in range(HALF_VREGS)]
                for b in range(B):
                    buf = gb[b]
                    for j in range(HALF_VREGS):
                        sl = pl.ds(hbase + j * LANES, LANES)
                        buf[r, sl] = buf[r, sl] + tvs[j]
            return carry

        lax.fori_loop(0, SUB, row, 0)

    load_group(0)
    for g in range(NG):
        if g + 1 < NG:
            load_group(g + 1)
        wait_group_loads(g)
        compute_group(g)
        store_group(g)

    for g in range(NG):
        wait_group_stores(g)


def kernel(seq_a, seq_b, emb_table):
    mesh = plsc.VectorSubcoreMesh(core_axis_name="c", subcore_axis_name="s")
    f = pl.kernel(
        _sc_body,
        out_type=jax.ShapeDtypeStruct((B, T, D), jnp.float32),
        mesh=mesh,
        scratch_types=(
            [pltpu.VMEM((SUB, D), jnp.float32) for _ in range(NP)]
            + [pltpu.VMEM((SUB, D), jnp.float32) for _ in range(NP * B)]
            + [pltpu.SemaphoreType.DMA for _ in range(NP + 2 * NP * B)]
        ),
    )
    return f(seq_a, seq_b, emb_table)


# speedup vs baseline: 1.0594x; 1.0572x over previous
import jax
import jax.numpy as jnp
from jax import lax
from jax.experimental import pallas as pl
from jax.experimental.pallas import tpu as pltpu
from jax.experimental.pallas import tpu_sc as plsc

B, T_HALF, D = 4, 1024, 1024
T = 2 * T_HALF
NW = 32
ROWS_PER_W = T // NW
SUB = 8
NG = ROWS_PER_W // SUB
NP = 3
LANES = 16
HALF_VREGS = 32


def _sc_body(seq_a, seq_b, emb, out, *scratch):
    tabs = scratch[0:NP]
    bufs = scratch[NP:2 * NP]
    sem_t = scratch[2 * NP:3 * NP]
    sem_l = scratch[3 * NP:4 * NP]
    sem_o = scratch[4 * NP:5 * NP]

    cid = lax.axis_index("c")
    sid = lax.axis_index("s")
    wid = sid * 2 + cid
    half = wid // 16
    r0 = (wid % 16) * ROWS_PER_W
    g0 = wid * ROWS_PER_W

    def load_group(g):
        p = g % NP
        pltpu.async_copy(emb.at[pl.ds(g0 + SUB * g, SUB), :],
                         tabs[p], sem_t[p])

        @pl.when(half == 0)
        def _():
            pltpu.async_copy(seq_a.at[:, pl.ds(r0 + SUB * g, SUB), :],
                             bufs[p], sem_l[p])

        @pl.when(half == 1)
        def _():
            pltpu.async_copy(seq_b.at[:, pl.ds(r0 + SUB * g, SUB), :],
                             bufs[p], sem_l[p])

    def wait_group_loads(g):
        p = g % NP
        pltpu.make_async_copy(emb.at[pl.ds(g0 + SUB * g, SUB), :],
                              tabs[p], sem_t[p]).wait()
        pltpu.make_async_copy(seq_a.at[:, pl.ds(r0 + SUB * g, SUB), :],
                              bufs[p], sem_l[p]).wait()

    def store_group(g):
        p = g % NP
        pltpu.async_copy(bufs[p], out.at[:, pl.ds(g0 + SUB * g, SUB), :],
                         sem_o[p])

    def wait_group_stores(g):
        p = g % NP
        pltpu.make_async_copy(bufs[p], out.at[:, pl.ds(g0 + SUB * g, SUB), :],
                              sem_o[p]).wait()

    def compute_group(g):
        p = g % NP
        tab = tabs[p]
        buf = bufs[p]

        def row(r, carry):
            for h in range(D // (HALF_VREGS * LANES)):
                hbase = h * HALF_VREGS * LANES
                tvs = [tab[r, pl.ds(hbase + j * LANES, LANES)]
                       for j in range(HALF_VREGS)]
                for b in range(B):
                    for j in range(HALF_VREGS):
                        sl = pl.ds(hbase + j * LANES, LANES)
                        buf[b, r, sl] = buf[b, r, sl] + tvs[j]
            return carry

        lax.fori_loop(0, SUB, row, 0)

    load_group(0)
    for g in range(NG):
        if g + 1 < NG:
            load_group(g + 1)
        wait_group_loads(g)
        compute_group(g)
        store_group(g)

    for g in range(NG):
        wait_group_stores(g)


def kernel(seq_a, seq_b, emb_table):
    mesh = plsc.VectorSubcoreMesh(core_axis_name="c", subcore_axis_name="s")
    f = pl.kernel(
        _sc_body,
        out_type=jax.ShapeDtypeStruct((B, T, D), jnp.float32),
        mesh=mesh,
        scratch_types=(
            [pltpu.VMEM((SUB, D), jnp.float32) for _ in range(NP)]
            + [pltpu.VMEM((B, SUB, D), jnp.float32) for _ in range(NP)]
            + [pltpu.SemaphoreType.DMA for _ in range(3 * NP)]
        ),
    )
    return f(seq_a, seq_b, emb_table)


# speedup vs baseline: 1.0801x; 1.0196x over previous
import jax
import jax.numpy as jnp
from jax import lax
from jax.experimental import pallas as pl
from jax.experimental.pallas import tpu as pltpu
from jax.experimental.pallas import tpu_sc as plsc

B, T_HALF, D = 4, 1024, 1024
T = 2 * T_HALF
NW = 32
ROWS_PER_W = T // NW
SUB = 8
NG = ROWS_PER_W // SUB
NP = 3
LANES = 16
HALF_VREGS = 32


def _sc_body(seq_a, seq_b, emb, out, *scratch):
    tabs = scratch[0:NP]
    bufs = scratch[NP:2 * NP]
    sem_t = scratch[2 * NP:3 * NP]
    sem_l = scratch[3 * NP:4 * NP]
    sem_o = scratch[4 * NP:5 * NP]

    cid = lax.axis_index("c")
    sid = lax.axis_index("s")
    wid = sid * 2 + cid
    half = wid // 16
    r0 = (wid % 16) * ROWS_PER_W
    g0 = wid * ROWS_PER_W

    def load_group(g):
        p = g % NP
        pltpu.async_copy(emb.at[pl.ds(g0 + SUB * g, SUB), :],
                         tabs[p], sem_t[p])

        @pl.when(half == 0)
        def _():
            pltpu.async_copy(seq_a.at[:, pl.ds(r0 + SUB * g, SUB), :],
                             bufs[p], sem_l[p])

        @pl.when(half == 1)
        def _():
            pltpu.async_copy(seq_b.at[:, pl.ds(r0 + SUB * g, SUB), :],
                             bufs[p], sem_l[p])

    def wait_group_loads(g):
        p = g % NP
        pltpu.make_async_copy(emb.at[pl.ds(g0 + SUB * g, SUB), :],
                              tabs[p], sem_t[p]).wait()
        pltpu.make_async_copy(seq_a.at[:, pl.ds(r0 + SUB * g, SUB), :],
                              bufs[p], sem_l[p]).wait()

    def store_group(g):
        p = g % NP
        pltpu.async_copy(bufs[p], out.at[:, pl.ds(g0 + SUB * g, SUB), :],
                         sem_o[p])

    def wait_group_stores(g):
        p = g % NP
        pltpu.make_async_copy(bufs[p], out.at[:, pl.ds(g0 + SUB * g, SUB), :],
                              sem_o[p]).wait()

    def compute_group(g):
        p = g % NP
        tab = tabs[p]
        buf = bufs[p]

        def row(r, carry):
            for h in range(D // (HALF_VREGS * LANES)):
                hbase = h * HALF_VREGS * LANES
                tvs = [tab[r, pl.ds(hbase + j * LANES, LANES)]
                       for j in range(HALF_VREGS)]
                for b in range(B):
                    for j in range(HALF_VREGS):
                        sl = pl.ds(hbase + j * LANES, LANES)
                        buf[b, r, sl] = buf[b, r, sl] + tvs[j]
            return carry

        lax.fori_loop(0, SUB, row, 0)

    load_group(0)
    load_group(1)
    for g in range(NG):
        if g + 2 < NG:
            load_group(g + 2)
        wait_group_loads(g)
        compute_group(g)
        store_group(g)

    for g in range(NG):
        wait_group_stores(g)


def kernel(seq_a, seq_b, emb_table):
    mesh = plsc.VectorSubcoreMesh(core_axis_name="c", subcore_axis_name="s")
    f = pl.kernel(
        _sc_body,
        out_type=jax.ShapeDtypeStruct((B, T, D), jnp.float32),
        mesh=mesh,
        scratch_types=(
            [pltpu.VMEM((SUB, D), jnp.float32) for _ in range(NP)]
            + [pltpu.VMEM((B, SUB, D), jnp.float32) for _ in range(NP)]
            + [pltpu.SemaphoreType.DMA for _ in range(3 * NP)]
        ),
    )
    return f(seq_a, seq_b, emb_table)
